# baseline (speedup 1.0000x reference)
import functools
import jax
import jax.numpy as jnp
from jax import lax
from jax.experimental import pallas as pl
from jax.experimental.pallas import tpu as pltpu
from jax.experimental.pallas import tpu_sc as plsc

N_ENT = 1000000
N_REL = 1000
D = 64
HD = 32
B = 16384
EPS = 1e-12

NC = 2
NS = 16
L = 16
NW = NC * NS
BPW = B // NW
CHUNK = 128
NCHUNK = BPW // CHUNK


def _trig_body(ph_ref, cs_ref):
    ph = ph_ref[...]
    cs_ref[...] = jnp.concatenate([jnp.cos(ph), jnp.sin(ph)], axis=1)


def _make_cs(phases):
    return pl.pallas_call(
        _trig_body,
        out_shape=jax.ShapeDtypeStruct((N_REL, D), jnp.float32),
    )(phases)


def _fast_sqrt(x):
    y = lax.bitcast_convert_type(x, jnp.int32)
    y = jnp.int32(0x5F3759DF) - lax.shift_right_logical(y, 1)
    y = lax.bitcast_convert_type(y, jnp.float32)
    for _ in range(3):
        y = y * (1.5 - 0.5 * x * y * y)
    return x * y


_MESH = plsc.VectorSubcoreMesh(core_axis_name="c", subcore_axis_name="s")


@functools.partial(
    pl.kernel,
    mesh=_MESH,
    compiler_params=pltpu.CompilerParams(
        needs_layout_passes=False, use_tc_tiling_on_sc=False
    ),
    out_type=jax.ShapeDtypeStruct((B,), jnp.float32),
    scratch_types=[
        pltpu.VMEM((NCHUNK, CHUNK), jnp.int32),
        pltpu.VMEM((NCHUNK, CHUNK), jnp.int32),
        pltpu.VMEM((NCHUNK, CHUNK), jnp.int32),
        pltpu.VMEM((BPW, D), jnp.float32),
        pltpu.VMEM((BPW, D), jnp.float32),
        pltpu.VMEM((BPW, D), jnp.float32),
        pltpu.VMEM((BPW,), jnp.float32),
        pltpu.SemaphoreType.DMA,
    ],
)
def _sc_kernel(h_hbm, t_hbm, r_hbm, ent_hbm, cs_hbm, out_hbm,
               hi_v, ti_v, ri_v, hrows, trows, csrows, out_v, sem):
    wid = lax.axis_index("s") * NC + lax.axis_index("c")

    pltpu.sync_copy(h_hbm.at[wid], hi_v)
    pltpu.sync_copy(t_hbm.at[wid], ti_v)
    pltpu.sync_copy(r_hbm.at[wid], ri_v)

    copies = []
    for c in range(NCHUNK):
        sl = pl.ds(c * CHUNK, CHUNK)
        copies.append(pltpu.async_copy(ent_hbm.at[hi_v.at[c]], hrows.at[sl], sem))
        copies.append(pltpu.async_copy(ent_hbm.at[ti_v.at[c]], trows.at[sl], sem))
        copies.append(pltpu.async_copy(cs_hbm.at[ri_v.at[c]], csrows.at[sl], sem))
    for cp in copies:
        cp.wait()

    lane = lax.iota(jnp.int32, L)

    def body(g, carry):
        row = g * L + lane
        acc = jnp.zeros((L,), jnp.float32)
        for j in range(HD):
            cj = jnp.full((L,), j, jnp.int32)
            hre = plsc.load_gather(hrows, [row, cj])
            him = plsc.load_gather(hrows, [row, cj + HD])
            tre = plsc.load_gather(trows, [row, cj])
            tim = plsc.load_gather(trows, [row, cj + HD])
            cc = plsc.load_gather(csrows, [row, cj])
            ss = plsc.load_gather(csrows, [row, cj + HD])
            dre = hre * cc - him * ss - tre
            dim = hre * ss + him * cc - tim
            acc = acc + _fast_sqrt(dre * dre + dim * dim + EPS)
        out_v[pl.ds(g * L, L)] = -acc
        return carry

    lax.fori_loop(0, BPW // L, body, 0)

    pltpu.sync_copy(out_v, out_hbm.at[pl.ds(wid * BPW, BPW)])


def kernel(h, r, t, entity_embed, relation_phases):
    cs = _make_cs(relation_phases)
    h3 = h.astype(jnp.int32).reshape(NW, NCHUNK, CHUNK)
    t3 = t.astype(jnp.int32).reshape(NW, NCHUNK, CHUNK)
    r3 = r.astype(jnp.int32).reshape(NW, NCHUNK, CHUNK)
    return _sc_kernel(h3, t3, r3, entity_embed, cs)


# speedup vs baseline: 1.0619x; 1.0619x over previous
import functools
import jax
import jax.numpy as jnp
from jax import lax
from jax.experimental import pallas as pl
from jax.experimental.pallas import tpu as pltpu
from jax.experimental.pallas import tpu_sc as plsc

N_ENT = 1000000
N_REL = 1000
D = 64
HD = 32
B = 16384
EPS = 1e-12
W = 128

NC = 2
NS = 16
L = 16
NW = NC * NS
BPW = B // NW
CHUNK = 128
NCHUNK = BPW // CHUNK
GPC = CHUNK // L
NBUF = 2


def _trig_body(ph_ref, cs_ref):
    ph = ph_ref[...]
    z = jnp.zeros_like(ph)
    cs_ref[...] = jnp.concatenate([jnp.cos(ph), jnp.sin(ph), z, z], axis=1)


def _make_cs(phases):
    return pl.pallas_call(
        _trig_body,
        out_shape=jax.ShapeDtypeStruct((N_REL, W), jnp.float32),
    )(phases)


def _fast_sqrt(x):
    y = lax.bitcast_convert_type(x, jnp.int32)
    y = jnp.int32(0x5F3759DF) - lax.shift_right_logical(y, 1)
    y = lax.bitcast_convert_type(y, jnp.float32)
    for _ in range(3):
        y = y * (1.5 - 0.5 * x * y * y)
    return x * y


_MESH = plsc.VectorSubcoreMesh(core_axis_name="c", subcore_axis_name="s")


@functools.partial(
    pl.kernel,
    mesh=_MESH,
    compiler_params=pltpu.CompilerParams(
        needs_layout_passes=False, use_tc_tiling_on_sc=True
    ),
    out_type=jax.ShapeDtypeStruct((B,), jnp.float32),
    scratch_types=[
        pltpu.VMEM((NCHUNK, CHUNK), jnp.int32),
        pltpu.VMEM((NCHUNK, CHUNK), jnp.int32),
        pltpu.VMEM((NCHUNK, CHUNK), jnp.int32),
        pltpu.VMEM((NCHUNK, CHUNK), jnp.int32),
        pltpu.VMEM((NCHUNK, CHUNK), jnp.int32),
        pltpu.VMEM((NBUF, CHUNK, W), jnp.float32),
        pltpu.VMEM((NBUF, CHUNK, W), jnp.float32),
        pltpu.VMEM((NBUF, CHUNK, W), jnp.float32),
        pltpu.VMEM((BPW,), jnp.float32),
        pltpu.SemaphoreType.DMA((NCHUNK,)),
    ],
)
def _sc_kernel(h_hbm, t_hbm, r_hbm, ent_hbm, cs_hbm, out_hbm,
               hi_v, ti_v, ri_v, ho_v, to_v, hbuf, tbuf, csbuf, out_v, sems):
    wid = lax.axis_index("s") * NC + lax.axis_index("c")

    pltpu.sync_copy(h_hbm.at[wid], hi_v)
    pltpu.sync_copy(t_hbm.at[wid], ti_v)
    pltpu.sync_copy(r_hbm.at[wid], ri_v)

    for c in range(NCHUNK):
        for v in range(CHUNK // L):
            sl = pl.ds(v * L, L)
            hi = hi_v[c, sl]
            ti = ti_v[c, sl]
            ho_v[c, sl] = jnp.bitwise_and(hi, 1) * D
            to_v[c, sl] = jnp.bitwise_and(ti, 1) * D
            hi_v[c, sl] = lax.shift_right_logical(hi, 1)
            ti_v[c, sl] = lax.shift_right_logical(ti, 1)

    copies = {}

    def fire(c):
        bi = c % NBUF
        copies[c] = (
            pltpu.async_copy(ent_hbm.at[hi_v.at[c]], hbuf.at[bi], sems.at[c]),
            pltpu.async_copy(ent_hbm.at[ti_v.at[c]], tbuf.at[bi], sems.at[c]),
            pltpu.async_copy(cs_hbm.at[ri_v.at[c]], csbuf.at[bi], sems.at[c]),
        )

    fire(0)
    fire(1)

    lane = lax.iota(jnp.int32, L)
    perms = [
        jnp.bitwise_and(lane + sh, L - 1).astype(jnp.int32) for sh in (8, 4, 2, 1)
    ]

    for c in range(NCHUNK):
        bi = c % NBUF
        for cp in copies[c]:
            cp.wait()

        def body(g, carry):
            hob = ho_v[c, pl.ds(g * L, L)]
            tob = to_v[c, pl.ds(g * L, L)]
            res = jnp.zeros((L,), jnp.float32)
            for e in range(L):
                ii = g * L + e
                hb = hob[e]
                tb = tob[e]
                acc = jnp.zeros((L,), jnp.float32)
                for k in range(HD // L):
                    hre = hbuf[bi, ii, pl.ds(hb + k * L, L)]
                    him = hbuf[bi, ii, pl.ds(hb + HD + k * L, L)]
                    tre = tbuf[bi, ii, pl.ds(tb + k * L, L)]
                    tim = tbuf[bi, ii, pl.ds(tb + HD + k * L, L)]
                    cc = csbuf[bi, ii, pl.ds(k * L, L)]
                    ss = csbuf[bi, ii, pl.ds(HD + k * L, L)]
                    dre = hre * cc - him * ss - tre
                    dim = hre * ss + him * cc - tim
                    acc = acc + _fast_sqrt(dre * dre + dim * dim + EPS)
                for p in perms:
                    acc = acc + jnp.take_along_axis(acc, p, axis=0)
                res = jnp.where(lane == e, acc, res)
            out_v[pl.ds((c * GPC + g) * L, L)] = jnp.zeros((L,), jnp.float32) - res
            return carry

        lax.fori_loop(0, GPC, body, 0)
        if c + NBUF < NCHUNK:
            fire(c + NBUF)

    pltpu.sync_copy(out_v, out_hbm.at[pl.ds(wid * BPW, BPW)])


def kernel(h, r, t, entity_embed, relation_phases):
    cs = _make_cs(relation_phases)
    ent2 = entity_embed.reshape(N_ENT // 2, W)
    h3 = h.astype(jnp.int32).reshape(NW, NCHUNK, CHUNK)
    t3 = t.astype(jnp.int32).reshape(NW, NCHUNK, CHUNK)
    r3 = r.astype(jnp.int32).reshape(NW, NCHUNK, CHUNK)
    return _sc_kernel(h3, t3, r3, ent2, cs)


# speedup vs baseline: 1.5847x; 1.4923x over previous
import functools
import jax
import jax.numpy as jnp
from jax import lax
from jax.experimental import pallas as pl
from jax.experimental.pallas import tpu as pltpu
from jax.experimental.pallas import tpu_sc as plsc

N_ENT = 1000000
N_REL = 1000
D = 64
HD = 32
B = 16384
EPS = 1e-12
W = 128

NC = 2
NS = 16
L = 16
NW = NC * NS
BPW = B // NW
CH = 16
NCH = BPW // CH
GPC = CH // L
NBUF = 2


def _trig_body(ph_ref, cs_ref):
    ph = ph_ref[...]
    z = jnp.zeros_like(ph)
    cs_ref[...] = jnp.concatenate([jnp.cos(ph), jnp.sin(ph), z, z], axis=1)


def _make_cs(phases):
    return pl.pallas_call(
        _trig_body,
        out_shape=jax.ShapeDtypeStruct((N_REL, W), jnp.float32),
    )(phases)


def _fast_sqrt(x):
    y = lax.bitcast_convert_type(x, jnp.int32)
    y = jnp.int32(0x5F3759DF) - lax.shift_right_logical(y, 1)
    y = lax.bitcast_convert_type(y, jnp.float32)
    for _ in range(3):
        y = y * (1.5 - 0.5 * x * y * y)
    return x * y


_MESH = plsc.VectorSubcoreMesh(core_axis_name="c", subcore_axis_name="s")


@functools.partial(
    pl.kernel,
    mesh=_MESH,
    compiler_params=pltpu.CompilerParams(
        needs_layout_passes=False, use_tc_tiling_on_sc=True
    ),
    out_type=jax.ShapeDtypeStruct((B,), jnp.float32),
    scratch_types=[
        pltpu.VMEM((BPW + L,), jnp.int32),
        pltpu.VMEM((BPW + L,), jnp.int32),
        pltpu.VMEM((NCH, CH), jnp.int32),
        pltpu.VMEM((NBUF, CH * 8, D), jnp.float32),
        pltpu.VMEM((NBUF, CH * 8, D), jnp.float32),
        pltpu.VMEM((NBUF, CH, W), jnp.float32),
        pltpu.VMEM((BPW,), jnp.float32),
        pltpu.SemaphoreType.DMA((NBUF,)),
        pltpu.SemaphoreType.DMA((NBUF,)),
    ],
)
def _sc_kernel(h_hbm, t_hbm, r_hbm, ent_hbm, cs_hbm, out_hbm,
               hv, tv, rv, hstage, tstage, csstage, out_v, sems, csems):
    wid = lax.axis_index("s") * NC + lax.axis_index("c")

    pltpu.sync_copy(h_hbm.at[wid], hv.at[pl.ds(0, BPW)])
    pltpu.sync_copy(t_hbm.at[wid], tv.at[pl.ds(0, BPW)])
    pltpu.sync_copy(r_hbm.at[wid], rv)

    def fire(c):
        bi = jnp.bitwise_and(c, NBUF - 1)

        def fbody(b, carry):
            hi = hv[pl.ds(c * CH + b, L)][0]
            ti = tv[pl.ds(c * CH + b, L)][0]
            hblk = pl.multiple_of(lax.shift_right_logical(hi, 3) * 8, 8)
            tblk = pl.multiple_of(lax.shift_right_logical(ti, 3) * 8, 8)
            dst = pl.ds(pl.multiple_of(b * 8, 8), 8)
            pltpu.async_copy(
                ent_hbm.at[pl.ds(hblk, 8), :], hstage.at[bi, dst, :], sems.at[bi]
            )
            pltpu.async_copy(
                ent_hbm.at[pl.ds(tblk, 8), :], tstage.at[bi, dst, :], sems.at[bi]
            )
            return carry

        lax.fori_loop(0, CH, fbody, 0)
        pltpu.async_copy(cs_hbm.at[rv.at[c]], csstage.at[bi], csems.at[bi])

    fire(jnp.int32(0))
    fire(jnp.int32(1))

    lane = lax.iota(jnp.int32, L)
    perms = [
        jnp.bitwise_and(lane + sh, L - 1).astype(jnp.int32) for sh in (8, 4, 2, 1)
    ]

    def chunk_body(c, carry):
        bi = jnp.bitwise_and(c, NBUF - 1)
        pltpu.make_async_copy(
            ent_hbm.at[pl.ds(0, CH * 8), :], hstage.at[bi], sems.at[bi]
        ).wait()
        pltpu.make_async_copy(
            ent_hbm.at[pl.ds(0, CH * 8), :], tstage.at[bi], sems.at[bi]
        ).wait()
        pltpu.make_async_copy(
            cs_hbm.at[pl.ds(0, CH)], csstage.at[bi], csems.at[bi]
        ).wait()

        hsr = jnp.bitwise_and(hv[pl.ds(c * CH, L)], 7)
        tsr = jnp.bitwise_and(tv[pl.ds(c * CH, L)], 7)
        res = jnp.zeros((L,), jnp.float32)
        for e in range(L):
            hrow = e * 8 + hsr[e]
            trow = e * 8 + tsr[e]
            acc = jnp.zeros((L,), jnp.float32)
            for k in range(HD // L):
                hre = hstage[bi, hrow, pl.ds(k * L, L)]
                him = hstage[bi, hrow, pl.ds(HD + k * L, L)]
                tre = tstage[bi, trow, pl.ds(k * L, L)]
                tim = tstage[bi, trow, pl.ds(HD + k * L, L)]
                cc = csstage[bi, e, pl.ds(k * L, L)]
                ss = csstage[bi, e, pl.ds(HD + k * L, L)]
                dre = hre * cc - him * ss - tre
                dim = hre * ss + him * cc - tim
                acc = acc + _fast_sqrt(dre * dre + dim * dim + EPS)
            for p in perms:
                acc = acc + jnp.take_along_axis(acc, p, axis=0)
            res = jnp.where(lane == e, acc, res)
        out_v[pl.ds(c * CH, L)] = jnp.zeros((L,), jnp.float32) - res

        @pl.when(c + NBUF < NCH)
        def _():
            fire(c + NBUF)

        return carry

    lax.fori_loop(0, NCH, chunk_body, 0)

    pltpu.sync_copy(out_v, out_hbm.at[pl.ds(wid * BPW, BPW)])


def kernel(h, r, t, entity_embed, relation_phases):
    cs = _make_cs(relation_phases)
    h2 = h.astype(jnp.int32).reshape(NW, BPW)
    t2 = t.astype(jnp.int32).reshape(NW, BPW)
    r3 = r.astype(jnp.int32).reshape(NW, NCH, CH)
    return _sc_kernel(h2, t2, r3, entity_embed, cs)


# speedup vs baseline: 2.2956x; 1.4486x over previous
import functools
import jax
import jax.numpy as jnp
from jax import lax
from jax.experimental import pallas as pl
from jax.experimental.pallas import tpu as pltpu
from jax.experimental.pallas import tpu_sc as plsc

N_ENT = 1000000
N_REL = 1000
D = 64
HD = 32
B = 16384
EPS = 1e-12
W = 128
NBLK = N_ENT // 8

NC = 2
NS = 16
L = 16
NW = NC * NS
BPW = B // NW
CH = 16
NCH = BPW // CH
GPC = CH // L
NBUF = 2


def _trig_body(ph_ref, cs_ref):
    ph = ph_ref[...]
    z = jnp.zeros_like(ph)
    cs_ref[...] = jnp.concatenate([jnp.cos(ph), jnp.sin(ph), z, z], axis=1)


def _make_cs(phases):
    return pl.pallas_call(
        _trig_body,
        out_shape=jax.ShapeDtypeStruct((N_REL, W), jnp.float32),
    )(phases)


def _fast_sqrt(x):
    y = lax.bitcast_convert_type(x, jnp.int32)
    y = jnp.int32(0x5F3759DF) - lax.shift_right_logical(y, 1)
    y = lax.bitcast_convert_type(y, jnp.float32)
    for _ in range(3):
        y = y * (1.5 - 0.5 * x * y * y)
    return x * y


_MESH = plsc.VectorSubcoreMesh(core_axis_name="c", subcore_axis_name="s")


@functools.partial(
    pl.kernel,
    mesh=_MESH,
    compiler_params=pltpu.CompilerParams(
        needs_layout_passes=False, use_tc_tiling_on_sc=True
    ),
    out_type=jax.ShapeDtypeStruct((B,), jnp.float32),
    scratch_types=[
        pltpu.VMEM((BPW + L,), jnp.int32),
        pltpu.VMEM((BPW + L,), jnp.int32),
        pltpu.VMEM((NCH, CH), jnp.int32),
        pltpu.VMEM((NBUF, CH, 8, D), jnp.float32),
        pltpu.VMEM((NBUF, CH, 8, D), jnp.float32),
        pltpu.VMEM((NBUF, CH, W), jnp.float32),
        pltpu.VMEM((BPW,), jnp.float32),
        pltpu.SemaphoreType.DMA((NBUF,)),
        pltpu.SemaphoreType.DMA((NBUF,)),
    ],
)
def _sc_kernel(h_hbm, t_hbm, r_hbm, ent_hbm, cs_hbm, out_hbm,
               hv, tv, rv, hstage, tstage, csstage, out_v,
               sems, csems):
    wid = lax.axis_index("s") * NC + lax.axis_index("c")

    pltpu.sync_copy(h_hbm.at[wid], hv.at[pl.ds(0, BPW)])
    pltpu.sync_copy(t_hbm.at[wid], tv.at[pl.ds(0, BPW)])
    pltpu.sync_copy(r_hbm.at[wid], rv)

    def fire(c, bi):
        def fbody(b, carry):
            hi = hv[pl.ds(c * CH + b, L)][0]
            ti = tv[pl.ds(c * CH + b, L)][0]
            pltpu.async_copy(
                ent_hbm.at[lax.shift_right_logical(hi, 3)],
                hstage.at[bi, b],
                sems.at[bi],
            )
            pltpu.async_copy(
                ent_hbm.at[lax.shift_right_logical(ti, 3)],
                tstage.at[bi, b],
                sems.at[bi],
            )
            return carry

        lax.fori_loop(0, CH, fbody, 0)
        pltpu.async_copy(cs_hbm.at[rv.at[c]], csstage.at[bi], csems.at[bi])

    fire(jnp.int32(0), jnp.int32(0))
    fire(jnp.int32(1), jnp.int32(1))

    lane = lax.iota(jnp.int32, L)
    perms = [
        jnp.bitwise_and(lane + sh, L - 1).astype(jnp.int32) for sh in (8, 4, 2, 1)
    ]

    def chunk_body(c, carry):
        bi = jnp.bitwise_and(c, NBUF - 1)
        pltpu.make_async_copy(
            ent_hbm.at[pl.ds(0, CH)], hstage.at[bi], sems.at[bi]
        ).wait()
        pltpu.make_async_copy(
            ent_hbm.at[pl.ds(0, CH)], tstage.at[bi], sems.at[bi]
        ).wait()
        pltpu.make_async_copy(
            cs_hbm.at[pl.ds(0, CH)], csstage.at[bi], csems.at[bi]
        ).wait()

        def body(g, carry2):
            hsr = jnp.bitwise_and(hv[pl.ds(c * CH + g * L, L)], 7)
            tsr = jnp.bitwise_and(tv[pl.ds(c * CH + g * L, L)], 7)
            res = jnp.zeros((L,), jnp.float32)
            for e in range(L):
                b = g * L + e
                acc = jnp.zeros((L,), jnp.float32)
                for k in range(HD // L):
                    hre = hstage[bi, b, hsr[e], pl.ds(k * L, L)]
                    him = hstage[bi, b, hsr[e], pl.ds(HD + k * L, L)]
                    tre = tstage[bi, b, tsr[e], pl.ds(k * L, L)]
                    tim = tstage[bi, b, tsr[e], pl.ds(HD + k * L, L)]
                    cc = csstage[bi, b, pl.ds(k * L, L)]
                    ss = csstage[bi, b, pl.ds(HD + k * L, L)]
                    dre = hre * cc - him * ss - tre
                    dim = hre * ss + him * cc - tim
                    acc = acc + _fast_sqrt(dre * dre + dim * dim + EPS)
                for p in perms:
                    acc = acc + jnp.take_along_axis(acc, p, axis=0)
                res = jnp.where(lane == e, acc, res)
            out_v[pl.ds((c * GPC + g) * L, L)] = jnp.zeros((L,), jnp.float32) - res
            return carry2

        lax.fori_loop(0, GPC, body, 0)

        @pl.when(c + NBUF < NCH)
        def _():
            fire(c + NBUF, bi)

        return carry

    lax.fori_loop(0, NCH, chunk_body, 0)

    pltpu.sync_copy(out_v, out_hbm.at[pl.ds(wid * BPW, BPW)])


def kernel(h, r, t, entity_embed, relation_phases):
    cs = _make_cs(relation_phases)
    ent3 = entity_embed.reshape(NBLK, 8, D)
    h2 = h.astype(jnp.int32).reshape(NW, BPW)
    t2 = t.astype(jnp.int32).reshape(NW, BPW)
    r3 = r.astype(jnp.int32).reshape(NW, NCH, CH)
    return _sc_kernel(h2, t2, r3, ent3, cs)


# speedup vs baseline: 2.3267x; 1.0135x over previous
import functools
import jax
import jax.numpy as jnp
from jax import lax
from jax.experimental import pallas as pl
from jax.experimental.pallas import tpu as pltpu
from jax.experimental.pallas import tpu_sc as plsc

N_ENT = 1000000
N_REL = 1000
D = 64
HD = 32
B = 16384
EPS = 1e-12
W = 128
NBLK = N_ENT // 8

NC = 2
NS = 16
L = 16
NW = NC * NS
BPW = B // NW
CH = 16
NCH = BPW // CH
GPC = CH // L
NBUF = 3


def _trig_body(ph_ref, cs_ref):
    ph = ph_ref[...]
    z = jnp.zeros_like(ph)
    cs_ref[...] = jnp.concatenate([jnp.cos(ph), jnp.sin(ph), z, z], axis=1)


def _make_cs(phases):
    return pl.pallas_call(
        _trig_body,
        out_shape=jax.ShapeDtypeStruct((N_REL, W), jnp.float32),
    )(phases)


def _fast_sqrt(x):
    y = lax.bitcast_convert_type(x, jnp.int32)
    y = jnp.int32(0x5F3759DF) - lax.shift_right_logical(y, 1)
    y = lax.bitcast_convert_type(y, jnp.float32)
    for _ in range(3):
        y = y * (1.5 - 0.5 * x * y * y)
    return x * y


_MESH = plsc.VectorSubcoreMesh(core_axis_name="c", subcore_axis_name="s")


@functools.partial(
    pl.kernel,
    mesh=_MESH,
    compiler_params=pltpu.CompilerParams(
        needs_layout_passes=False, use_tc_tiling_on_sc=True
    ),
    out_type=jax.ShapeDtypeStruct((B,), jnp.float32),
    scratch_types=[
        pltpu.VMEM((BPW + L,), jnp.int32),
        pltpu.VMEM((BPW + L,), jnp.int32),
        pltpu.VMEM((NCH, CH), jnp.int32),
        pltpu.VMEM((NBUF, CH, 8, D), jnp.float32),
        pltpu.VMEM((NBUF, CH, 8, D), jnp.float32),
        pltpu.VMEM((NBUF, CH, W), jnp.float32),
        pltpu.VMEM((BPW,), jnp.float32),
        pltpu.SemaphoreType.DMA((NBUF,)),
        pltpu.SemaphoreType.DMA((NBUF,)),
    ],
)
def _sc_kernel(h_hbm, t_hbm, r_hbm, ent_hbm, cs_hbm, out_hbm,
               hv, tv, rv, hstage, tstage, csstage, out_v,
               sems, csems):
    wid = lax.axis_index("s") * NC + lax.axis_index("c")

    pltpu.sync_copy(h_hbm.at[wid], hv.at[pl.ds(0, BPW)])
    pltpu.sync_copy(t_hbm.at[wid], tv.at[pl.ds(0, BPW)])
    pltpu.sync_copy(r_hbm.at[wid], rv)

    def fire(c, bi):
        def fbody(b, carry):
            hi = hv[pl.ds(c * CH + b, L)][0]
            ti = tv[pl.ds(c * CH + b, L)][0]
            pltpu.async_copy(
                ent_hbm.at[lax.shift_right_logical(hi, 3)],
                hstage.at[bi, b],
                sems.at[bi],
            )
            pltpu.async_copy(
                ent_hbm.at[lax.shift_right_logical(ti, 3)],
                tstage.at[bi, b],
                sems.at[bi],
            )
            return carry

        lax.fori_loop(0, CH, fbody, 0)
        pltpu.async_copy(cs_hbm.at[rv.at[c]], csstage.at[bi], csems.at[bi])

    for c0 in range(NBUF):
        fire(jnp.int32(c0), jnp.int32(c0))

    lane = lax.iota(jnp.int32, L)
    perms = [
        jnp.bitwise_and(lane + sh, L - 1).astype(jnp.int32) for sh in (8, 4, 2, 1)
    ]

    def chunk_body(c, carry):
        bi = lax.rem(c, jnp.int32(NBUF))
        pltpu.make_async_copy(
            ent_hbm.at[pl.ds(0, CH)], hstage.at[bi], sems.at[bi]
        ).wait()
        pltpu.make_async_copy(
            ent_hbm.at[pl.ds(0, CH)], tstage.at[bi], sems.at[bi]
        ).wait()
        pltpu.make_async_copy(
            cs_hbm.at[pl.ds(0, CH)], csstage.at[bi], csems.at[bi]
        ).wait()

        def body(g, carry2):
            hsr = jnp.bitwise_and(hv[pl.ds(c * CH + g * L, L)], 7)
            tsr = jnp.bitwise_and(tv[pl.ds(c * CH + g * L, L)], 7)
            res = jnp.zeros((L,), jnp.float32)
            for e in range(L):
                b = g * L + e
                acc = jnp.zeros((L,), jnp.float32)
                for k in range(HD // L):
                    hre = hstage[bi, b, hsr[e], pl.ds(k * L, L)]
                    him = hstage[bi, b, hsr[e], pl.ds(HD + k * L, L)]
                    tre = tstage[bi, b, tsr[e], pl.ds(k * L, L)]
                    tim = tstage[bi, b, tsr[e], pl.ds(HD + k * L, L)]
                    cc = csstage[bi, b, pl.ds(k * L, L)]
                    ss = csstage[bi, b, pl.ds(HD + k * L, L)]
                    dre = hre * cc - him * ss - tre
                    dim = hre * ss + him * cc - tim
                    acc = acc + _fast_sqrt(dre * dre + dim * dim + EPS)
                for p in perms:
                    acc = acc + jnp.take_along_axis(acc, p, axis=0)
                res = jnp.where(lane == e, acc, res)
            out_v[pl.ds((c * GPC + g) * L, L)] = jnp.zeros((L,), jnp.float32) - res
            return carry2

        lax.fori_loop(0, GPC, body, 0)

        @pl.when(c + NBUF < NCH)
        def _():
            fire(c + NBUF, bi)

        return carry

    lax.fori_loop(0, NCH, chunk_body, 0)

    pltpu.sync_copy(out_v, out_hbm.at[pl.ds(wid * BPW, BPW)])


def kernel(h, r, t, entity_embed, relation_phases):
    cs = _make_cs(relation_phases)
    ent3 = entity_embed.reshape(NBLK, 8, D)
    h2 = h.astype(jnp.int32).reshape(NW, BPW)
    t2 = t.astype(jnp.int32).reshape(NW, BPW)
    r3 = r.astype(jnp.int32).reshape(NW, NCH, CH)
    return _sc_kernel(h2, t2, r3, ent3, cs)
